# Initial kernel scaffold; baseline (speedup 1.0000x reference)
import jax
import jax.numpy as jnp
from jax.experimental import pallas as pl


def kernel(x, edge_index, root_index, batch, W1, b1, W2, b2):
    raise NotImplementedError("write your pallas kernel here")


# speedup vs baseline: 11.4596x; 11.4596x over previous
import functools

import jax
import jax.numpy as jnp
from jax import lax
from jax.experimental import pallas as pl
from jax.experimental.pallas import tpu as pltpu
from jax.experimental.pallas import tpu_sc as plsc

NN = 10000
EE = 320000
FF = 128
GG = 64
NP_ = 10240
KC = 128
NTILES = 32
CPT = 79
EPAD = NTILES * CPT * KC
RB = 1000
NBLK = NN // RB


def _mesh():
    return plsc.VectorSubcoreMesh(core_axis_name="c", subcore_axis_name="s")


def _deg_body(dst_hbm, out_hbm, idx_v, ones_v, stage_v, acc):
    c = lax.axis_index("c")
    s = lax.axis_index("s")
    wid = s * 2 + c

    def fill_z(i, _):
        stage_v[pl.ds(i * 16, 16)] = jnp.zeros((16,), jnp.float32)
        return 0
    lax.fori_loop(0, 40, fill_z, 0)

    def fill_o(i, _):
        ones_v[pl.ds(i * 16, 16)] = jnp.full((16,), 1.0, jnp.float32)
        return 0
    lax.fori_loop(0, 8, fill_o, 0)

    pltpu.sync_copy(stage_v, acc.at[pl.ds(s * 640, 640)])
    plsc.subcore_barrier()

    def step(t, _):
        off = (wid * CPT + t) * KC
        pltpu.sync_copy(dst_hbm.at[pl.ds(off, KC)], idx_v)
        pltpu.sync_copy(ones_v, acc.at[idx_v], add=True)
        return 0
    lax.fori_loop(0, CPT, step, 0)

    plsc.subcore_barrier()
    pltpu.sync_copy(acc.at[pl.ds(s * 640, 640)], stage_v)
    pltpu.sync_copy(stage_v, out_hbm.at[pl.ds(c * NP_ + s * 640, 640)])


@jax.jit
def _sc_deg(dst_p):
    k = pl.kernel(
        _deg_body,
        out_type=jax.ShapeDtypeStruct((2 * NP_,), jnp.float32),
        mesh=_mesh(),
        scratch_types=[
            pltpu.VMEM((KC,), jnp.int32),
            pltpu.VMEM((KC,), jnp.float32),
            pltpu.VMEM((640,), jnp.float32),
            pltpu.VMEM_SHARED((NP_,), jnp.float32),
        ],
    )
    return k(dst_p)


def _spmm_body(g_hbm, src_hbm, dst_hbm, out_hbm, si, di, rows, zrow, acc):
    c = lax.axis_index("c")
    s = lax.axis_index("s")
    wid = s * 2 + c

    def fill_z(i, _):
        r = i // 8
        l = i - r * 8
        zrow[r, pl.ds(l * 16, 16)] = jnp.zeros((16,), jnp.float32)
        return 0
    lax.fori_loop(0, 512, fill_z, 0)

    def zstripe(t, _):
        pltpu.sync_copy(zrow, acc.at[pl.ds(s * 640 + t * 64, 64)])
        return 0
    lax.fori_loop(0, 10, zstripe, 0)
    plsc.subcore_barrier()

    def step(t, _):
        off = (wid * CPT + t) * KC
        pltpu.sync_copy(src_hbm.at[pl.ds(off, KC)], si)
        pltpu.sync_copy(dst_hbm.at[pl.ds(off, KC)], di)
        pltpu.sync_copy(g_hbm.at[si], rows)
        pltpu.sync_copy(rows, acc.at[di], add=True)
        return 0
    lax.fori_loop(0, CPT, step, 0)

    plsc.subcore_barrier()

    def wout(t, _):
        pltpu.sync_copy(acc.at[pl.ds(s * 640 + t * 128, 128)], rows)
        pltpu.sync_copy(rows, out_hbm.at[pl.ds(c * NP_ + s * 640 + t * 128, 128)])
        return 0
    lax.fori_loop(0, 5, wout, 0)


@jax.jit
def _sc_spmm(g, src_p, dst_p):
    k = pl.kernel(
        _spmm_body,
        out_type=jax.ShapeDtypeStruct((2 * NP_, FF), jnp.float32),
        mesh=_mesh(),
        scratch_types=[
            pltpu.VMEM((KC,), jnp.int32),
            pltpu.VMEM((KC,), jnp.int32),
            pltpu.VMEM((KC, FF), jnp.float32),
            pltpu.VMEM((64, FF), jnp.float32),
            pltpu.VMEM_SHARED((NP_, FF), jnp.float32),
        ],
    )
    return k(g, src_p, dst_p)


def _tca_body(x_ref, w1_ref, d0_ref, d1_ref, rid_ref,
              h1_ref, g1_ref, dinv_ref, r1_ref):
    i = pl.program_id(0)
    xb = x_ref[...]
    h1 = jnp.dot(xb, w1_ref[...], preferred_element_type=jnp.float32)
    deg = d0_ref[...] + d1_ref[...] + 1.0
    dinv = lax.rsqrt(deg)
    h1_ref[...] = h1
    dinv_ref[...] = dinv
    g1_ref[...] = h1 * dinv
    rid = rid_ref[...]
    col = lax.broadcasted_iota(jnp.int32, (GG, RB), 1) + i * RB
    pmat = (rid == col).astype(jnp.float32)

    @pl.when(i == 0)
    def _():
        r1_ref[...] = jnp.zeros((GG, FF), jnp.float32)

    r1_ref[...] += jnp.dot(pmat, xb, preferred_element_type=jnp.float32)


@jax.jit
def _tc_a(x, W1, deg0, deg1, rid):
    return pl.pallas_call(
        _tca_body,
        grid=(NBLK,),
        in_specs=[
            pl.BlockSpec((RB, FF), lambda i: (i, 0)),
            pl.BlockSpec((FF, FF), lambda i: (0, 0)),
            pl.BlockSpec((RB, 1), lambda i: (i, 0)),
            pl.BlockSpec((RB, 1), lambda i: (i, 0)),
            pl.BlockSpec((GG, 1), lambda i: (0, 0)),
        ],
        out_specs=[
            pl.BlockSpec((RB, FF), lambda i: (i, 0)),
            pl.BlockSpec((RB, FF), lambda i: (i, 0)),
            pl.BlockSpec((RB, 1), lambda i: (i, 0)),
            pl.BlockSpec((GG, FF), lambda i: (0, 0)),
        ],
        out_shape=[
            jax.ShapeDtypeStruct((NN, FF), jnp.float32),
            jax.ShapeDtypeStruct((NN, FF), jnp.float32),
            jax.ShapeDtypeStruct((NN, 1), jnp.float32),
            jax.ShapeDtypeStruct((GG, FF), jnp.float32),
        ],
    )(x, W1, deg0, deg1, rid)


def _tcb_body(a0_ref, a1_ref, h1_ref, dinv_ref, b1_ref, bat_ref, rid_ref,
              r1_ref, w2a_ref, w2b_ref,
              g2_ref, h2l_ref, r2_ref, r1w_ref):
    i = pl.program_id(0)

    @pl.when(i == 0)
    def _():
        r1w_ref[...] = jnp.dot(jnp.maximum(r1_ref[...], 0.0), w2b_ref[...],
                               preferred_element_type=jnp.float32)
        r2_ref[...] = jnp.zeros((GG, FF), jnp.float32)

    dinv = dinv_ref[...]
    c1 = dinv * (a0_ref[...] + a1_ref[...]) + dinv * dinv * h1_ref[...] + b1_ref[...]
    relu1 = jnp.maximum(c1, 0.0)
    bat = bat_ref[...]
    bmat = (bat == lax.broadcasted_iota(jnp.int32, (RB, GG), 1)).astype(jnp.float32)
    h2 = (jnp.dot(relu1, w2a_ref[...], preferred_element_type=jnp.float32)
          + jnp.dot(bmat, r1w_ref[...], preferred_element_type=jnp.float32))
    h2l_ref[...] = h2
    g2_ref[...] = h2 * dinv

    rid = rid_ref[...]
    col = lax.broadcasted_iota(jnp.int32, (GG, RB), 1) + i * RB
    pmat = (rid == col).astype(jnp.float32)
    r2_ref[...] += jnp.dot(pmat, c1, preferred_element_type=jnp.float32)


@jax.jit
def _tc_b(a0, a1, h1, dinv, b1, bat, rid, roots1, w2a, w2b):
    return pl.pallas_call(
        _tcb_body,
        grid=(NBLK,),
        in_specs=[
            pl.BlockSpec((RB, FF), lambda i: (i, 0)),
            pl.BlockSpec((RB, FF), lambda i: (i, 0)),
            pl.BlockSpec((RB, FF), lambda i: (i, 0)),
            pl.BlockSpec((RB, 1), lambda i: (i, 0)),
            pl.BlockSpec((1, FF), lambda i: (0, 0)),
            pl.BlockSpec((RB, 1), lambda i: (i, 0)),
            pl.BlockSpec((GG, 1), lambda i: (0, 0)),
            pl.BlockSpec((GG, FF), lambda i: (0, 0)),
            pl.BlockSpec((FF, FF), lambda i: (0, 0)),
            pl.BlockSpec((FF, FF), lambda i: (0, 0)),
        ],
        out_specs=[
            pl.BlockSpec((RB, FF), lambda i: (i, 0)),
            pl.BlockSpec((RB, FF), lambda i: (i, 0)),
            pl.BlockSpec((GG, FF), lambda i: (0, 0)),
        ],
        out_shape=[
            jax.ShapeDtypeStruct((NN, FF), jnp.float32),
            jax.ShapeDtypeStruct((NN, FF), jnp.float32),
            jax.ShapeDtypeStruct((GG, FF), jnp.float32),
        ],
        scratch_shapes=[pltpu.VMEM((GG, FF), jnp.float32)],
    )(a0, a1, h1, dinv, b1, bat, rid, roots1, w2a, w2b)


def _tcc_body(a0_ref, a1_ref, h2l_ref, dinv_ref, b2_ref, bat_ref, r2_ref,
              out_ref, sum_ref, cnt_ref):
    i = pl.program_id(0)

    @pl.when(i == 0)
    def _():
        sum_ref[...] = jnp.zeros((GG, FF), jnp.float32)
        cnt_ref[...] = jnp.zeros((GG, 1), jnp.float32)

    dinv = dinv_ref[...]
    c2 = dinv * (a0_ref[...] + a1_ref[...]) + dinv * dinv * h2l_ref[...] + b2_ref[...]
    relu2 = jnp.maximum(c2, 0.0)
    bat = bat_ref[...]
    bmat = (bat == lax.broadcasted_iota(jnp.int32, (RB, GG), 1)).astype(jnp.float32)
    sum_ref[...] += lax.dot_general(bmat, relu2, (((0,), (0,)), ((), ())),
                                    preferred_element_type=jnp.float32)
    ones_col = jnp.ones((RB, 1), jnp.float32)
    cnt_ref[...] += lax.dot_general(bmat, ones_col, (((0,), (0,)), ((), ())),
                                    preferred_element_type=jnp.float32)

    @pl.when(i == NBLK - 1)
    def _():
        cnt = cnt_ref[...]
        mean = sum_ref[...] / jnp.maximum(cnt, 1.0)
        rootp = jnp.where(cnt > 0.0, r2_ref[...], 0.0)
        out_ref[...] = jnp.concatenate([mean, rootp], axis=1)


@jax.jit
def _tc_c(a0, a1, h2l, dinv, b2, bat, roots2):
    return pl.pallas_call(
        _tcc_body,
        grid=(NBLK,),
        in_specs=[
            pl.BlockSpec((RB, FF), lambda i: (i, 0)),
            pl.BlockSpec((RB, FF), lambda i: (i, 0)),
            pl.BlockSpec((RB, FF), lambda i: (i, 0)),
            pl.BlockSpec((RB, 1), lambda i: (i, 0)),
            pl.BlockSpec((1, FF), lambda i: (0, 0)),
            pl.BlockSpec((RB, 1), lambda i: (i, 0)),
            pl.BlockSpec((GG, FF), lambda i: (0, 0)),
        ],
        out_specs=pl.BlockSpec((GG, 2 * FF), lambda i: (0, 0)),
        out_shape=jax.ShapeDtypeStruct((GG, 2 * FF), jnp.float32),
        scratch_shapes=[
            pltpu.VMEM((GG, FF), jnp.float32),
            pltpu.VMEM((GG, 1), jnp.float32),
        ],
    )(a0, a1, h2l, dinv, b2, bat, roots2)


def kernel(x, edge_index, root_index, batch, W1, b1, W2, b2):
    x = x.astype(jnp.float32)
    src = edge_index[0]
    dst = edge_index[1]
    npad = EPAD - EE
    src_p = jnp.concatenate([src, jnp.zeros((npad,), jnp.int32)])
    dst_p = jnp.concatenate(
        [dst, NN + (jnp.arange(npad, dtype=jnp.int32) % (NP_ - NN))])

    degf = _sc_deg(dst_p)
    deg0 = degf[:NP_].reshape(NP_, 1)
    deg1 = degf[NP_:].reshape(NP_, 1)
    rid = root_index.reshape(GG, 1)
    bat = batch.reshape(NN, 1)

    h1, g1, dinv, roots1 = _tc_a(x, W1, deg0, deg1, rid)

    p1 = _sc_spmm(g1, src_p, dst_p)
    g2, h2l, roots2 = _tc_b(p1[:NP_], p1[NP_:], h1, dinv,
                            b1.reshape(1, FF), bat, rid, roots1,
                            W2[:FF], W2[FF:])

    p2 = _sc_spmm(g2, src_p, dst_p)
    out = _tc_c(p2[:NP_], p2[NP_:], h2l, dinv, b2.reshape(1, FF), bat, roots2)
    return out
